# baseline (speedup 1.0000x reference)
import jax
import jax.numpy as jnp
import numpy as np
from jax import lax
from jax.experimental import pallas as pl
from jax.experimental.pallas import tpu as pltpu
from jax.experimental.pallas import tpu_sc as plsc

N = 10000
E = 320000
D = 128
H = 4
DH = 32
INV_SQRT_DH = np.float32(1.0 / np.sqrt(DH))

NC = 2
NS = 16
EDGES_PER_CORE = E // NC
EDGES_PER_TILE = EDGES_PER_CORE // NS
CHUNK = 40
N_CHUNKS = EDGES_PER_TILE // CHUNK
NP = 10240
ROWS_PER_TILE = NP // NS
ZROWS = 128
AW = 144


def _node_qkv_body(x_ref, wq_ref, wk_ref, wv_ref, g_ref, b_ref, qt_ref, kt_ref,
                   vt_ref):
    x = x_ref[...]
    m = jnp.mean(x, axis=0, keepdims=True)
    v = jnp.mean((x - m) ** 2, axis=0, keepdims=True)
    xn = (x - m) * lax.rsqrt(v + 1e-5) * g_ref[0:1, :] + b_ref[0:1, :]
    qt_ref[...] = jnp.dot(xn, wq_ref[...].T, preferred_element_type=jnp.float32)
    k = jnp.dot(xn, wk_ref[...].T, preferred_element_type=jnp.float32)
    kt_ref[...] = k * INV_SQRT_DH
    vt_ref[...] = jnp.dot(xn, wv_ref[...].T, preferred_element_type=jnp.float32)


def _node_qkv(x, wq, wk, wv, g, b):
    g8 = jnp.broadcast_to(g[None, :], (8, D))
    b8 = jnp.broadcast_to(b[None, :], (8, D))
    out = jax.ShapeDtypeStruct((N, D), jnp.float32)
    return pl.pallas_call(
        _node_qkv_body,
        out_shape=(out, out, out),
    )(x, wq, wk, wv, g8, b8)


EBLK = 2000
N_EBLK = E // EBLK


def _edge_stats_body(e_ref, s_ref, q_ref):
    i = pl.program_id(0)
    blk = e_ref[...]
    ps = jnp.sum(blk, axis=0, keepdims=True)
    pq = jnp.sum(blk * blk, axis=0, keepdims=True)
    ps8 = jnp.broadcast_to(ps, (8, D))
    pq8 = jnp.broadcast_to(pq, (8, D))

    @pl.when(i == 0)
    def _():
        s_ref[...] = ps8
        q_ref[...] = pq8

    @pl.when(i > 0)
    def _():
        s_ref[...] += ps8
        q_ref[...] += pq8


def _edge_stats(e):
    return pl.pallas_call(
        _edge_stats_body,
        grid=(N_EBLK,),
        in_specs=[pl.BlockSpec((EBLK, D), lambda i: (i, 0))],
        out_specs=(pl.BlockSpec((8, D), lambda i: (0, 0)),
                   pl.BlockSpec((8, D), lambda i: (0, 0))),
        out_shape=(jax.ShapeDtypeStruct((8, D), jnp.float32),
                   jax.ShapeDtypeStruct((8, D), jnp.float32)),
    )(e)


def _proj_e_body(e_ref, w_ref, b_ref, o_ref):
    p = jnp.dot(e_ref[...], w_ref[...].T, preferred_element_type=jnp.float32)
    o_ref[...] = p + b_ref[0:1, :]


def _proj_e(e, w_eff, b_eff):
    b8 = jnp.broadcast_to(b_eff[None, :], (8, D))
    return pl.pallas_call(
        _proj_e_body,
        grid=(N_EBLK,),
        in_specs=[pl.BlockSpec((EBLK, D), lambda i: (i, 0)),
                  pl.BlockSpec((D, D), lambda i: (0, 0)),
                  pl.BlockSpec((8, D), lambda i: (0, 0))],
        out_specs=pl.BlockSpec((EBLK, D), lambda i: (i, 0)),
        out_shape=jax.ShapeDtypeStruct((E, D), jnp.float32),
    )(e, w_eff, b8)


ZR = 384
ZSTRIPE = ZR // NS
ZC = CHUNK + 8


def _sc_edge_body(qt, kt, vt, pe, src, dst, out, zout,
                  src_v, dst_v, dstpad_v, zrow_v, krows, qrows, vrows, pe_v,
                  contrib, zrec, acc, z_sh, sem):
    c = lax.axis_index("c")
    s = lax.axis_index("s")
    lanes = lax.iota(jnp.int32, 16)
    zero16 = jnp.zeros((16,), jnp.float32)

    def _zero_crow(i, _):
        for j in range(D // 16):
            contrib[i, pl.ds(16 * j, 16)] = zero16
        return 0

    def _zero_zrow(i, _):
        for j in range(D // 16):
            zrec[i, pl.ds(16 * j, 16)] = zero16
        return 0

    lax.fori_loop(0, CHUNK, _zero_crow, 0)
    lax.fori_loop(0, ZC, _zero_zrow, 0)
    for k in range(ROWS_PER_TILE // CHUNK):
        pltpu.sync_copy(contrib, acc.at[pl.ds(s * ROWS_PER_TILE + k * CHUNK,
                                              CHUNK)])
    pltpu.sync_copy(contrib.at[pl.ds(0, ZSTRIPE)],
                    z_sh.at[pl.ds(s * ZSTRIPE, ZSTRIPE)])
    plsc.subcore_barrier()

    def _lane_sum(x):
        for sh in (8, 4, 2, 1):
            x = x + jnp.take_along_axis(x, lanes ^ sh, axis=0)
        return x

    def _chunk(i, _):
        base = c * EDGES_PER_CORE + s * EDGES_PER_TILE + i * CHUNK
        pltpu.sync_copy(src.at[pl.ds(base, CHUNK)], src_v)
        pltpu.sync_copy(dst.at[pl.ds(base, CHUNK)], dst_v)
        pltpu.sync_copy(dst.at[pl.ds(base, CHUNK)],
                        dstpad_v.at[pl.ds(0, CHUNK)])
        g1 = pltpu.async_copy(kt.at[src_v], krows, sem)
        g2 = pltpu.async_copy(vt.at[src_v], vrows, sem)
        g3 = pltpu.async_copy(qt.at[dst_v], qrows, sem)
        g4 = pltpu.async_copy(pe.at[pl.ds(base, CHUNK)], pe_v, sem)
        for j in range(ZC // 16):
            dvj = dstpad_v[pl.ds(16 * j, 16)]
            r = jax.lax.shift_right_logical(dvj, 5)
            if 16 * (j + 1) > CHUNK:
                r = jnp.where(lanes < CHUNK - 16 * j, r, ZR - 1)
            zrow_v[pl.ds(16 * j, 16)] = r
        g1.wait()
        g2.wait()
        g3.wait()
        g4.wait()

        def _edge(e, _):
            zv = zero16
            for h in range(H):
                ka = krows[e, pl.ds(32 * h, 16)]
                kb = krows[e, pl.ds(32 * h + 16, 16)]
                qa = qrows[e, pl.ds(32 * h, 16)]
                qb = qrows[e, pl.ds(32 * h + 16, 16)]
                pa = pe_v[e, pl.ds(32 * h, 16)]
                pb = pe_v[e, pl.ds(32 * h + 16, 16)]
                ta = jnp.clip(ka * qa, -5.0, 5.0) * pa
                tb = jnp.clip(kb * qb, -5.0, 5.0) * pb
                w = jnp.exp(jnp.clip(_lane_sum(ta + tb), -5.0, 5.0))
                contrib[e, pl.ds(32 * h, 16)] = vrows[e, pl.ds(32 * h, 16)] * w
                contrib[e, pl.ds(32 * h + 16, 16)] = (
                    vrows[e, pl.ds(32 * h + 16, 16)] * w)
                zv = jnp.where(lanes == h, w, zv)
            dv = dstpad_v[pl.ds(e, 16)]
            zcol = jnp.broadcast_to(
                jax.lax.bitwise_and(dv[0], 31) * 4, (16,)) + lanes
            erow = jnp.broadcast_to(e, (16,))
            plsc.store_scatter(zrec, [erow, zcol], zv, mask=lanes < 4)
            return 0

        lax.fori_loop(0, CHUNK, _edge, 0)
        pltpu.sync_copy(contrib, acc.at[dst_v], add=True)
        pltpu.sync_copy(zrec, z_sh.at[zrow_v], add=True)

        def _edge_clear(e, _):
            dv = dstpad_v[pl.ds(e, 16)]
            zcol = jnp.broadcast_to(
                jax.lax.bitwise_and(dv[0], 31) * 4, (16,)) + lanes
            erow = jnp.broadcast_to(e, (16,))
            plsc.store_scatter(zrec, [erow, zcol], zero16, mask=lanes < 4)
            return 0

        lax.fori_loop(0, CHUNK, _edge_clear, 0)
        return 0

    lax.fori_loop(0, N_CHUNKS, _chunk, 0)
    plsc.subcore_barrier()

    for k in range(ROWS_PER_TILE // CHUNK):
        r0 = s * ROWS_PER_TILE + k * CHUNK
        pltpu.sync_copy(acc.at[pl.ds(r0, CHUNK)],
                        out.at[c].at[pl.ds(r0, CHUNK)])
    pltpu.sync_copy(z_sh.at[pl.ds(s * ZSTRIPE, ZSTRIPE)],
                    zout.at[c].at[pl.ds(s * ZSTRIPE, ZSTRIPE)])


def _sc_edge(qt, kt, vt, pe, src, dst):
    mesh = plsc.VectorSubcoreMesh(core_axis_name="c", subcore_axis_name="s",
                                  num_cores=NC, num_subcores=NS)
    f = pl.kernel(
        _sc_edge_body,
        out_type=(jax.ShapeDtypeStruct((NC, NP, D), jnp.float32),
                  jax.ShapeDtypeStruct((NC, ZR, D), jnp.float32)),
        mesh=mesh,
        compiler_params=pltpu.CompilerParams(needs_layout_passes=False),
        scratch_types=[
            pltpu.VMEM((CHUNK,), jnp.int32),
            pltpu.VMEM((CHUNK,), jnp.int32),
            pltpu.VMEM((CHUNK + 16,), jnp.int32),
            pltpu.VMEM((ZC,), jnp.int32),
            pltpu.VMEM((CHUNK, D), jnp.float32),
            pltpu.VMEM((CHUNK, D), jnp.float32),
            pltpu.VMEM((CHUNK, D), jnp.float32),
            pltpu.VMEM((CHUNK, D), jnp.float32),
            pltpu.VMEM((CHUNK, D), jnp.float32),
            pltpu.VMEM((ZC, D), jnp.float32),
            pltpu.VMEM_SHARED((NP, D), jnp.float32),
            pltpu.VMEM_SHARED((ZR, D), jnp.float32),
            pltpu.SemaphoreType.DMA,
        ],
    )
    return f(qt, kt, vt, pe, src, dst)


def _final_body(acc_ref, zt_ref, x1_ref, wo_ref, bo_ref, w1_ref, w2_ref,
                g2_ref, b2_ref, wih_ref, whh_ref, bih_ref, bhh_ref, out_ref):
    wv = acc_ref[0][0:N] + acc_ref[1][0:N]
    zt = zt_ref[...]
    den = jnp.concatenate([
        jnp.broadcast_to(zt[0:N, h:h + 1], (N, DH)) for h in range(H)
    ], axis=1) + 1e-6
    h = wv / den
    h = jnp.dot(h, wo_ref[...].T, preferred_element_type=jnp.float32)
    h = h + bo_ref[0:1, :]
    x = x1_ref[...] + h
    x_in2 = x
    m = jnp.mean(x, axis=0, keepdims=True)
    v = jnp.mean((x - m) ** 2, axis=0, keepdims=True)
    xn = (x - m) * lax.rsqrt(v + 1e-5) * g2_ref[0:1, :] + b2_ref[0:1, :]
    y = jnp.dot(xn, w1_ref[...].T, preferred_element_type=jnp.float32)
    y = y * jax.nn.sigmoid(y)
    y = jnp.dot(y, w2_ref[...].T, preferred_element_type=jnp.float32)
    x = x_in2 + y

    wih_t = wih_ref[...].T
    whh_t = whh_ref[...].T
    bih = bih_ref[0:1, :]
    bhh = bhh_ref[0:1, :]
    q_star = jnp.zeros((1, 2 * D), jnp.float32)
    hh = jnp.zeros((1, D), jnp.float32)
    cc = jnp.zeros((1, D), jnp.float32)
    for _ in range(3):
        gates = (jnp.dot(q_star, wih_t, preferred_element_type=jnp.float32)
                 + bih
                 + jnp.dot(hh, whh_t, preferred_element_type=jnp.float32)
                 + bhh)
        ig = jax.nn.sigmoid(gates[:, 0:D])
        fg = jax.nn.sigmoid(gates[:, D:2 * D])
        gg = jnp.tanh(gates[:, 2 * D:3 * D])
        og = jax.nn.sigmoid(gates[:, 3 * D:4 * D])
        cc = fg * cc + ig * gg
        hh = og * jnp.tanh(cc)
        logits = jnp.sum(x * hh, axis=1, keepdims=True)
        lmax = jnp.max(logits, axis=0, keepdims=True)
        ex = jnp.exp(logits - lmax)
        alpha = ex / jnp.sum(ex, axis=0, keepdims=True)
        r = jnp.sum(alpha * x, axis=0, keepdims=True)
        q_star = jnp.concatenate([hh, r], axis=1)
    out_ref[...] = q_star


def _final(acc, zt, x1, wo, bo, w1, w2, g2, b2, wih, whh, bih, bhh):
    bo8 = jnp.broadcast_to(bo[None, :], (8, D))
    g28 = jnp.broadcast_to(g2[None, :], (8, D))
    b28 = jnp.broadcast_to(b2[None, :], (8, D))
    bih8 = jnp.broadcast_to(bih[None, :], (8, 4 * D))
    bhh8 = jnp.broadcast_to(bhh[None, :], (8, 4 * D))
    return pl.pallas_call(
        _final_body,
        out_shape=jax.ShapeDtypeStruct((1, 2 * D), jnp.float32),
    )(acc, zt, x1, wo, bo8, w1, w2, g28, b28, wih, whh, bih8, bhh8)


def kernel(node_feats, edge_feats, edge_index, Wq, Wk, Wv, We, Wo, bo, W1, W2,
           g1n, b1n, g1e, b1e, g2, b2, Wih, Whh, bih, bhh):
    src = edge_index[0].astype(jnp.int32)
    dst = edge_index[1].astype(jnp.int32)

    qt, kt, vt = _node_qkv(node_feats, Wq, Wk, Wv, g1n, b1n)

    ssum, ssq = _edge_stats(edge_feats)
    mean_e = ssum[0] / E
    var_e = ssq[0] / E - mean_e * mean_e
    s_e = g1e * lax.rsqrt(var_e + 1e-5)
    we_eff = We * s_e[None, :]
    be_eff = (b1e - mean_e * s_e) @ We.T
    pe = _proj_e(edge_feats, we_eff, be_eff)

    acc, z_out = _sc_edge(qt, kt, vt, pe, src, dst)
    zt = (z_out[0] + z_out[1]).reshape(ZR * D)[:4 * NP].reshape(NP, 4)

    return _final(acc, zt, node_feats, Wo, bo, W1, W2, g2, b2, Wih, Whh, bih,
                  bhh)
